# Initial kernel scaffold; baseline (speedup 1.0000x reference)
import jax
import jax.numpy as jnp
from jax import lax
from jax.experimental import pallas as pl
from jax.experimental.pallas import tpu as pltpu
from jax.experimental.pallas import tpu_sc as plsc

VOCAB = 1000000
MAXLEN = 200
EMBED_DIM = 32
BATCH = 4096
SEQ = 200

NC = 2
NS = 16
NW = NC * NS

SEQ_PER_W = BATCH // NW
CS = 4
CHUNKS = SEQ_PER_W // CS
ROWS_PER_CHUNK = CS * SEQ
IDX_ROW = 100
IDX_ROWS_PER_CHUNK = ROWS_PER_CHUNK // IDX_ROW
X_ROWS = BATCH * SEQ // IDX_ROW


def _sc_kernel(x_hbm, tok_hbm, pos_hbm, out_hbm, idx_v, dst_v, pos_v, sem_g, sem_s):
    wid = lax.axis_index("s") * NC + lax.axis_index("c")
    base_idx_row = wid * (SEQ_PER_W * SEQ // IDX_ROW)
    base_out_row = wid * SEQ_PER_W * SEQ

    pltpu.sync_copy(pos_hbm, pos_v)

    def chunk_body(c, carry):
        pltpu.sync_copy(
            x_hbm.at[pl.ds(base_idx_row + c * IDX_ROWS_PER_CHUNK,
                           IDX_ROWS_PER_CHUNK)],
            idx_v)
        for u in range(IDX_ROWS_PER_CHUNK):
            pltpu.async_copy(
                tok_hbm.at[idx_v.at[u]],
                dst_v.at[pl.ds(u * IDX_ROW, IDX_ROW)],
                sem_g)
        for u in range(IDX_ROWS_PER_CHUNK):
            pltpu.make_async_copy(
                tok_hbm.at[idx_v.at[u]],
                dst_v.at[pl.ds(u * IDX_ROW, IDX_ROW)],
                sem_g).wait()

        def add_body(r, carry2):
            for h in range(EMBED_DIM // 16):
                pvec = pos_v[r, pl.ds(h * 16, 16)]
                for s in range(CS):
                    plsc.addupdate(dst_v.at[s * SEQ + r, pl.ds(h * 16, 16)],
                                   pvec)
            return carry2
        lax.fori_loop(0, SEQ, add_body, 0)

        pltpu.async_copy(
            dst_v,
            out_hbm.at[pl.ds(base_out_row + c * ROWS_PER_CHUNK,
                             ROWS_PER_CHUNK)],
            sem_s).wait()
        return carry

    lax.fori_loop(0, CHUNKS, chunk_body, 0)


def kernel(x, token_table, pos_table):
    x2 = x.reshape(X_ROWS, IDX_ROW).astype(jnp.int32)
    mesh = plsc.VectorSubcoreMesh(core_axis_name="c", subcore_axis_name="s",
                                  num_cores=NC, num_subcores=NS)
    out_flat = pl.kernel(
        _sc_kernel,
        out_type=jax.ShapeDtypeStruct((BATCH * SEQ, EMBED_DIM), jnp.float32),
        mesh=mesh,
        scratch_types=[
            pltpu.VMEM((IDX_ROWS_PER_CHUNK, IDX_ROW), jnp.int32),
            pltpu.VMEM((ROWS_PER_CHUNK, EMBED_DIM), jnp.float32),
            pltpu.VMEM((MAXLEN, EMBED_DIM), jnp.float32),
            pltpu.SemaphoreType.DMA,
            pltpu.SemaphoreType.DMA,
        ],
    )(x2, token_table, pos_table)
    return out_flat.reshape(BATCH, SEQ, EMBED_DIM)


# speedup vs baseline: 1.3894x; 1.3894x over previous
import jax
import jax.numpy as jnp
from jax import lax
from jax.experimental import pallas as pl
from jax.experimental.pallas import tpu as pltpu
from jax.experimental.pallas import tpu_sc as plsc

VOCAB = 1000000
MAXLEN = 200
EMBED_DIM = 32
BATCH = 4096
SEQ = 200

NC = 2
NS = 16
NW = NC * NS

SEQ_PER_W = BATCH // NW
CS = 4
CHUNKS = SEQ_PER_W // CS
ROWS_PER_CHUNK = CS * SEQ
IDX_ROW = 100
IDX_ROWS_PER_CHUNK = ROWS_PER_CHUNK // IDX_ROW
X_ROWS = BATCH * SEQ // IDX_ROW


def _sc_kernel(x_hbm, tok_hbm, pos_hbm, out_hbm, idx_v, dst_v, pos_v, sem_g, sem_s):
    wid = lax.axis_index("s") * NC + lax.axis_index("c")
    base_idx_row = wid * (SEQ_PER_W * SEQ // IDX_ROW)
    base_out_row = wid * SEQ_PER_W * SEQ

    pltpu.sync_copy(pos_hbm, pos_v)

    def chunk_body(c, carry):
        pltpu.sync_copy(
            x_hbm.at[pl.ds(base_idx_row + c * IDX_ROWS_PER_CHUNK,
                           IDX_ROWS_PER_CHUNK)],
            idx_v)
        for u in range(IDX_ROWS_PER_CHUNK):
            pltpu.async_copy(
                tok_hbm.at[idx_v.at[u]],
                dst_v.at[pl.ds(u * IDX_ROW, IDX_ROW)],
                sem_g)
        for u in range(IDX_ROWS_PER_CHUNK):
            pltpu.make_async_copy(
                tok_hbm.at[idx_v.at[u]],
                dst_v.at[pl.ds(u * IDX_ROW, IDX_ROW)],
                sem_g).wait()

        def add_body(r, carry2):
            for h in range(EMBED_DIM // 16):
                pvec = pos_v[r, pl.ds(h * 16, 16)]
                for s in range(CS):
                    plsc.addupdate(dst_v.at[s * SEQ + r, pl.ds(h * 16, 16)],
                                   pvec)
            return carry2
        lax.fori_loop(0, SEQ, add_body, 0)

        pltpu.async_copy(
            dst_v,
            out_hbm.at[pl.ds(base_out_row + c * ROWS_PER_CHUNK,
                             ROWS_PER_CHUNK)],
            sem_s).wait()
        return carry

    lax.fori_loop(0, CHUNKS, chunk_body, 0)


def kernel(x, token_table, pos_table):
    x2 = x.reshape(X_ROWS, IDX_ROW).astype(jnp.int32)
    mesh = plsc.VectorSubcoreMesh(core_axis_name="c", subcore_axis_name="s",
                                  num_cores=NC, num_subcores=NS)
    out_flat = pl.kernel(
        _sc_kernel,
        out_type=jax.ShapeDtypeStruct((BATCH * SEQ, EMBED_DIM), jnp.float32),
        mesh=mesh,
        compiler_params=pltpu.CompilerParams(use_tc_tiling_on_sc=False),
        scratch_types=[
            pltpu.VMEM((IDX_ROWS_PER_CHUNK, IDX_ROW), jnp.int32),
            pltpu.VMEM((ROWS_PER_CHUNK, EMBED_DIM), jnp.float32),
            pltpu.VMEM((MAXLEN, EMBED_DIM), jnp.float32),
            pltpu.SemaphoreType.DMA,
            pltpu.SemaphoreType.DMA,
        ],
    )(x2, token_table, pos_table)
    return out_flat.reshape(BATCH, SEQ, EMBED_DIM)


# speedup vs baseline: 1.4933x; 1.0748x over previous
import jax
import jax.numpy as jnp
from jax import lax
from jax.experimental import pallas as pl
from jax.experimental.pallas import tpu as pltpu
from jax.experimental.pallas import tpu_sc as plsc

VOCAB = 1000000
MAXLEN = 200
EMBED_DIM = 32
BATCH = 4096
SEQ = 200

NC = 2
NS = 16
NW = NC * NS

SEQ_PER_W = BATCH // NW
CS = 2
CHUNKS = SEQ_PER_W // CS
ROWS_PER_CHUNK = CS * SEQ
IDX_ROW = 100
STREAMS = ROWS_PER_CHUNK // IDX_ROW
IDX_ROWS_W = SEQ_PER_W * SEQ // IDX_ROW
X_ROWS = BATCH * SEQ // IDX_ROW
NBUF = 4


def _sc_kernel(x_hbm, tok_hbm, pos_hbm, out_hbm, idx_v, dst_v, pos_v, *sems):
    sem_g = sems[:NBUF]
    sem_s = sems[NBUF:]
    wid = lax.axis_index("s") * NC + lax.axis_index("c")
    base_idx_row = wid * IDX_ROWS_W
    base_out_row = wid * SEQ_PER_W * SEQ

    pltpu.sync_copy(pos_hbm, pos_v)
    pltpu.sync_copy(x_hbm.at[pl.ds(base_idx_row, IDX_ROWS_W)], idx_v)

    def fire_gathers(c, b):
        for u in range(STREAMS):
            pltpu.async_copy(
                tok_hbm.at[idx_v.at[c * STREAMS + u]],
                dst_v.at[pl.ds(b * ROWS_PER_CHUNK + u * IDX_ROW, IDX_ROW)],
                sem_g[b])

    def drain_gathers(c, b):
        for u in range(STREAMS):
            pltpu.make_async_copy(
                tok_hbm.at[idx_v.at[c * STREAMS + u]],
                dst_v.at[pl.ds(b * ROWS_PER_CHUNK + u * IDX_ROW, IDX_ROW)],
                sem_g[b]).wait()

    def add_pos(b):
        def add_body(r, carry):
            for h in range(EMBED_DIM // 16):
                pvec = pos_v[r, pl.ds(h * 16, 16)]
                for s in range(CS):
                    plsc.addupdate(
                        dst_v.at[b * ROWS_PER_CHUNK + s * SEQ + r,
                                 pl.ds(h * 16, 16)],
                        pvec)
            return carry
        lax.fori_loop(0, SEQ, add_body, 0)

    def fire_scatter(c, b):
        pltpu.async_copy(
            dst_v.at[pl.ds(b * ROWS_PER_CHUNK, ROWS_PER_CHUNK)],
            out_hbm.at[pl.ds(base_out_row + c * ROWS_PER_CHUNK,
                             ROWS_PER_CHUNK)],
            sem_s[b])

    def drain_scatter(c, b):
        pltpu.make_async_copy(
            dst_v.at[pl.ds(b * ROWS_PER_CHUNK, ROWS_PER_CHUNK)],
            out_hbm.at[pl.ds(base_out_row + c * ROWS_PER_CHUNK,
                             ROWS_PER_CHUNK)],
            sem_s[b]).wait()

    fire_gathers(0, 0)
    fire_gathers(1, 1)
    for c in range(2):
        drain_gathers(c, c)
        add_pos(c)
        fire_scatter(c, c)
        fire_gathers(c + 2, c + 2)

    def main_body(i, carry):
        for j in range(NBUF):
            c = 2 + i * NBUF + j
            b = (2 + j) % NBUF
            drain_scatter(c - 2, (b + 2) % NBUF)
            fire_gathers(c + 2, (b + 2) % NBUF)
            drain_gathers(c, b)
            add_pos(b)
            fire_scatter(c, b)
        return carry
    lax.fori_loop(0, (CHUNKS - 4) // NBUF, main_body, 0)

    for k in range(2):
        c = CHUNKS - 2 + k
        b = c % NBUF
        drain_scatter(c - 2, (b + 2) % NBUF)
        drain_gathers(c, b)
        add_pos(b)
        fire_scatter(c, b)
    for k in range(2):
        c = CHUNKS - 2 + k
        drain_scatter(c, c % NBUF)


def kernel(x, token_table, pos_table):
    x2 = x.reshape(X_ROWS, IDX_ROW).astype(jnp.int32)
    mesh = plsc.VectorSubcoreMesh(core_axis_name="c", subcore_axis_name="s",
                                  num_cores=NC, num_subcores=NS)
    out_flat = pl.kernel(
        _sc_kernel,
        out_type=jax.ShapeDtypeStruct((BATCH * SEQ, EMBED_DIM), jnp.float32),
        mesh=mesh,
        compiler_params=pltpu.CompilerParams(use_tc_tiling_on_sc=False),
        scratch_types=[
            pltpu.VMEM((IDX_ROWS_W, IDX_ROW), jnp.int32),
            pltpu.VMEM((NBUF * ROWS_PER_CHUNK, EMBED_DIM), jnp.float32),
            pltpu.VMEM((MAXLEN, EMBED_DIM), jnp.float32),
        ] + [pltpu.SemaphoreType.DMA] * (2 * NBUF),
    )(x2, token_table, pos_table)
    return out_flat.reshape(BATCH, SEQ, EMBED_DIM)
